# baseline (speedup 1.0000x reference)
import functools

import jax
import jax.numpy as jnp
from jax import lax
from jax.experimental import pallas as pl
from jax.experimental.pallas import tpu as pltpu
from jax.experimental.pallas import tpu_sc as plsc

N = 10000
E = 160000
NFEAT = 128
NHID = 128
NCLASS = 64

NC = 2
NS = 16
NW = NC * NS
CHUNK = 64
CPT = 81
EPAD = NW * CPT * CHUNK
NP = 10112
RPT = NP // NS

_MESH = plsc.VectorSubcoreMesh(core_axis_name="c", subcore_axis_name="s")

MBLK = 2000


DEGW = 128


def _sc_degree(dst_tiles, ones_rows, zeros_deg):
    @functools.partial(
        pl.kernel,
        out_type=jax.ShapeDtypeStruct((NC, NP, DEGW), jnp.float32),
        mesh=_MESH,
        scratch_types=[
            pltpu.VMEM((CPT, CHUNK), jnp.int32),
            pltpu.VMEM((CHUNK, DEGW), jnp.float32),
            pltpu.VMEM_SHARED((NP, DEGW), jnp.float32),
            pltpu.SemaphoreType.DMA,
            pltpu.SemaphoreType.DMA,
        ],
    )
    def deg_kernel(dst_hbm, ones_hbm, zeros_hbm, out_hbm, idx_v, ones_v, acc_s,
                   s0, s1):
        cid = lax.axis_index("c")
        sid = lax.axis_index("s")
        gtile = cid * NS + sid
        base = sid * RPT
        pltpu.sync_copy(zeros_hbm.at[pl.ds(base, RPT)], acc_s.at[pl.ds(base, RPT)])
        pltpu.sync_copy(ones_hbm, ones_v)
        pltpu.sync_copy(dst_hbm.at[gtile], idx_v)
        plsc.subcore_barrier()

        @pl.loop(0, CPT)
        def _(j):
            pltpu.async_copy(ones_v, acc_s.at[idx_v.at[j]], s0, add=True)

        @pl.loop(0, CPT)
        def _(j):
            pltpu.make_async_copy(ones_v, acc_s.at[idx_v.at[j]], s0).wait()

        plsc.subcore_barrier()
        pltpu.sync_copy(acc_s.at[pl.ds(base, RPT)],
                        out_hbm.at[cid, pl.ds(base, RPT)])

    return deg_kernel(dst_tiles, ones_rows, zeros_deg)


def _sc_edge_pass(hs, src_tiles, dst_tiles, zeros128):
    NBUF = 3
    NGRP = CPT // NBUF
    assert CPT % NBUF == 0

    @functools.partial(
        pl.kernel,
        out_type=jax.ShapeDtypeStruct((NC, NP, NHID), jnp.float32),
        mesh=_MESH,
        scratch_types=(
            [pltpu.VMEM((CPT, CHUNK), jnp.int32),
             pltpu.VMEM((CPT, CHUNK), jnp.int32)]
            + [pltpu.VMEM((CHUNK, NHID), jnp.float32)] * NBUF
            + [pltpu.SemaphoreType.DMA] * (2 * NBUF)
            + [pltpu.VMEM_SHARED((NP, NHID), jnp.float32)]
        ),
    )
    def edge_kernel(hs_hbm, src_hbm, dst_hbm, zeros_hbm, out_hbm,
                    src_v, dst_v, *rest):
        bufs = rest[:NBUF]
        gsem = rest[NBUF:2 * NBUF]
        ssem = rest[2 * NBUF:3 * NBUF]
        acc_s = rest[3 * NBUF]
        cid = lax.axis_index("c")
        sid = lax.axis_index("s")
        gtile = cid * NS + sid
        base = sid * RPT
        pltpu.sync_copy(zeros_hbm.at[pl.ds(base, RPT)], acc_s.at[pl.ds(base, RPT)])
        pltpu.sync_copy(src_hbm.at[gtile], src_v)
        pltpu.sync_copy(dst_hbm.at[gtile], dst_v)
        plsc.subcore_barrier()

        def wait_gather(j, k):
            pltpu.make_async_copy(hs_hbm.at[src_v.at[j]], bufs[k], gsem[k]).wait()

        def wait_scatter(j, k):
            pltpu.make_async_copy(bufs[k], acc_s.at[dst_v.at[j]], ssem[k]).wait()

        for k in range(NBUF):
            pltpu.async_copy(hs_hbm.at[src_v.at[k]], bufs[k], gsem[k])

        @pl.loop(1, NGRP)
        def _(i):
            j0 = i * NBUF
            for k in range(NBUF):
                wait_gather(j0 - NBUF + k, k)
                pltpu.async_copy(bufs[k], acc_s.at[dst_v.at[j0 - NBUF + k]],
                                 ssem[k], add=True)
            for k in range(NBUF):
                wait_scatter(j0 - NBUF + k, k)
                pltpu.async_copy(hs_hbm.at[src_v.at[j0 + k]], bufs[k], gsem[k])

        for k in range(NBUF):
            wait_gather(CPT - NBUF + k, k)
            pltpu.async_copy(bufs[k], acc_s.at[dst_v.at[CPT - NBUF + k]],
                             ssem[k], add=True)
        for k in range(NBUF):
            wait_scatter(CPT - NBUF + k, k)

        plsc.subcore_barrier()
        pltpu.sync_copy(acc_s.at[pl.ds(base, RPT)],
                        out_hbm.at[cid, pl.ds(base, RPT)])

    return edge_kernel(hs, src_tiles, dst_tiles, zeros128)


def _dot(a, b):
    return jnp.dot(a, b, preferred_element_type=jnp.float32,
                   precision=lax.Precision.HIGHEST)


def _row_spec(cols):
    return pl.BlockSpec((MBLK, cols), lambda i: (i, 0))


def _full_spec(rows, cols):
    return pl.BlockSpec((rows, cols), lambda i: (0, 0))


def _tc_encode(x, enc_W, enc_b):
    def body(x_ref, w_ref, b_ref, o_ref):
        o_ref[...] = jnp.maximum(_dot(x_ref[...], w_ref[...]) + b_ref[...], 0.0)

    return pl.pallas_call(
        body,
        grid=(N // MBLK,),
        in_specs=[_row_spec(NFEAT), _full_spec(NFEAT, NHID), _full_spec(1, NHID)],
        out_specs=_row_spec(NHID),
        out_shape=jax.ShapeDtypeStruct((N, NHID), jnp.float32),
    )(x, enc_W, enc_b.reshape(1, NHID))


def _tc_scale_matmul(x0, conv_W, dega, degb):
    def body(x_ref, w_ref, da_ref, db_ref, o_ref):
        dinv = lax.rsqrt(da_ref[:, 0:1] + db_ref[:, 0:1] + 1.0)
        o_ref[...] = dinv * _dot(x_ref[...], w_ref[...])

    return pl.pallas_call(
        body,
        grid=(N // MBLK,),
        in_specs=[_row_spec(NHID), _full_spec(NHID, NHID),
                  _row_spec(16), _row_spec(16)],
        out_specs=_row_spec(NHID),
        out_shape=jax.ShapeDtypeStruct((N, NHID), jnp.float32),
    )(x0, conv_W, dega, degb)


def _tc_post_and_next(acc0, acc1, hs, x_prev, conv_b, conv_W, dega, degb):
    def body(a0_ref, a1_ref, hs_ref, xp_ref, b_ref, w_ref, da_ref, db_ref,
             x_ref, hsn_ref):
        dinv = lax.rsqrt(da_ref[:, 0:1] + db_ref[:, 0:1] + 1.0)
        agg = a0_ref[...] + a1_ref[...] + hs_ref[...]
        x_new = jnp.maximum(dinv * agg + b_ref[...] + xp_ref[...], 0.0)
        x_ref[...] = x_new
        hsn_ref[...] = dinv * _dot(x_new, w_ref[...])

    return pl.pallas_call(
        body,
        grid=(N // MBLK,),
        in_specs=[_row_spec(NHID), _row_spec(NHID), _row_spec(NHID),
                  _row_spec(NHID), _full_spec(1, NHID), _full_spec(NHID, NHID),
                  _row_spec(16), _row_spec(16)],
        out_specs=[_row_spec(NHID), _row_spec(NHID)],
        out_shape=[jax.ShapeDtypeStruct((N, NHID), jnp.float32),
                   jax.ShapeDtypeStruct((N, NHID), jnp.float32)],
    )(acc0, acc1, hs, x_prev, conv_b.reshape(1, NHID), conv_W, dega, degb)


def _tc_post_and_decode(acc0, acc1, hs, x_prev, conv_b, dec_W, dec_b, dega, degb):
    def body(a0_ref, a1_ref, hs_ref, xp_ref, b_ref, w_ref, db2_ref, da_ref,
             db_ref, o_ref):
        dinv = lax.rsqrt(da_ref[:, 0:1] + db_ref[:, 0:1] + 1.0)
        agg = a0_ref[...] + a1_ref[...] + hs_ref[...]
        x_new = jnp.maximum(dinv * agg + b_ref[...] + xp_ref[...], 0.0)
        o_ref[...] = _dot(x_new, w_ref[...]) + db2_ref[...]

    return pl.pallas_call(
        body,
        grid=(N // MBLK,),
        in_specs=[_row_spec(NHID), _row_spec(NHID), _row_spec(NHID),
                  _row_spec(NHID), _full_spec(1, NHID), _full_spec(NHID, NCLASS),
                  _full_spec(1, NCLASS), _row_spec(16), _row_spec(16)],
        out_specs=_row_spec(NCLASS),
        out_shape=jax.ShapeDtypeStruct((N, NCLASS), jnp.float32),
    )(acc0, acc1, hs, x_prev, conv_b.reshape(1, NHID), dec_W,
      dec_b.reshape(1, NCLASS), dega, degb)


def kernel(x, edge_index, enc_W, enc_b, conv_W, conv_b, dec_W, dec_b):
    src = edge_index[0]
    dst = edge_index[1]
    pad = EPAD - E
    srcp = jnp.concatenate([src, jnp.zeros((pad,), jnp.int32)])
    dstp = jnp.concatenate([dst, jnp.full((pad,), N, jnp.int32)])
    src_tiles = srcp.reshape(NW, CPT, CHUNK)
    dst_tiles = dstp.reshape(NW, CPT, CHUNK)

    zeros128 = jnp.zeros((NP, NHID), jnp.float32)
    ones_rows = jnp.ones((CHUNK, DEGW), jnp.float32)

    degs = _sc_degree(dst_tiles, ones_rows, zeros128)
    dega = degs[0, :N, :16]
    degb = degs[1, :N, :16]

    x0 = _tc_encode(x, enc_W, enc_b)
    hs1 = _tc_scale_matmul(x0, conv_W, dega, degb)

    acc = _sc_edge_pass(hs1, src_tiles, dst_tiles, zeros128)
    x1, hs2 = _tc_post_and_next(acc[0, :N], acc[1, :N], hs1, x0,
                                conv_b, conv_W, dega, degb)

    acc2 = _sc_edge_pass(hs2, src_tiles, dst_tiles, zeros128)
    out = _tc_post_and_decode(acc2[0, :N], acc2[1, :N], hs2, x1,
                              conv_b, dec_W, dec_b, dega, degb)
    return out


# speedup vs baseline: 1.3451x; 1.3451x over previous
import functools

import jax
import jax.numpy as jnp
from jax import lax
from jax.experimental import pallas as pl
from jax.experimental.pallas import tpu as pltpu
from jax.experimental.pallas import tpu_sc as plsc

N = 10000
E = 160000
NFEAT = 128
NHID = 128
NCLASS = 64

NC = 2
NS = 16
NW = NC * NS
CHUNK = 128
CPT = 40
EPAD = NW * CPT * CHUNK
NP = 10112
RPT = NP // NS

_MESH = plsc.VectorSubcoreMesh(core_axis_name="c", subcore_axis_name="s")

MBLK = 2000


DEGW = 128


def _sc_degree(dst_tiles, ones_rows, zeros_deg):
    @functools.partial(
        pl.kernel,
        out_type=jax.ShapeDtypeStruct((NC, NP, DEGW), jnp.float32),
        mesh=_MESH,
        scratch_types=[
            pltpu.VMEM((CPT, CHUNK), jnp.int32),
            pltpu.VMEM((CHUNK, DEGW), jnp.float32),
            pltpu.VMEM_SHARED((NP, DEGW), jnp.float32),
            pltpu.SemaphoreType.DMA,
            pltpu.SemaphoreType.DMA,
        ],
    )
    def deg_kernel(dst_hbm, ones_hbm, zeros_hbm, out_hbm, idx_v, ones_v, acc_s,
                   s0, s1):
        cid = lax.axis_index("c")
        sid = lax.axis_index("s")
        gtile = cid * NS + sid
        base = sid * RPT
        pltpu.sync_copy(zeros_hbm.at[pl.ds(base, RPT)], acc_s.at[pl.ds(base, RPT)])
        pltpu.sync_copy(ones_hbm, ones_v)
        pltpu.sync_copy(dst_hbm.at[gtile], idx_v)
        plsc.subcore_barrier()

        @pl.loop(0, CPT)
        def _(j):
            pltpu.async_copy(ones_v, acc_s.at[idx_v.at[j]], s0, add=True)

        @pl.loop(0, CPT)
        def _(j):
            pltpu.make_async_copy(ones_v, acc_s.at[idx_v.at[j]], s0).wait()

        plsc.subcore_barrier()
        pltpu.sync_copy(acc_s.at[pl.ds(base, RPT)],
                        out_hbm.at[cid, pl.ds(base, RPT)])

    return deg_kernel(dst_tiles, ones_rows, zeros_deg)


def _sc_edge_pass(hs, src_tiles, dst_tiles, zeros128):
    NBUF = 2
    NGRP = CPT // NBUF
    assert CPT % NBUF == 0

    @functools.partial(
        pl.kernel,
        out_type=jax.ShapeDtypeStruct((NC, NP, NHID), jnp.float32),
        mesh=_MESH,
        scratch_types=(
            [pltpu.VMEM((CPT, CHUNK), jnp.int32),
             pltpu.VMEM((CPT, CHUNK), jnp.int32)]
            + [pltpu.VMEM((CHUNK, NHID), jnp.float32)] * NBUF
            + [pltpu.SemaphoreType.DMA] * (2 * NBUF)
            + [pltpu.VMEM_SHARED((NP, NHID), jnp.float32)]
        ),
    )
    def edge_kernel(hs_hbm, src_hbm, dst_hbm, zeros_hbm, out_hbm,
                    src_v, dst_v, *rest):
        bufs = rest[:NBUF]
        gsem = rest[NBUF:2 * NBUF]
        ssem = rest[2 * NBUF:3 * NBUF]
        acc_s = rest[3 * NBUF]
        cid = lax.axis_index("c")
        sid = lax.axis_index("s")
        gtile = cid * NS + sid
        base = sid * RPT
        pltpu.sync_copy(zeros_hbm.at[pl.ds(base, RPT)], acc_s.at[pl.ds(base, RPT)])
        pltpu.sync_copy(src_hbm.at[gtile], src_v)
        pltpu.sync_copy(dst_hbm.at[gtile], dst_v)
        plsc.subcore_barrier()

        def wait_gather(j, k):
            pltpu.make_async_copy(hs_hbm.at[src_v.at[j]], bufs[k], gsem[k]).wait()

        def wait_scatter(j, k):
            pltpu.make_async_copy(bufs[k], acc_s.at[dst_v.at[j]], ssem[k]).wait()

        for k in range(NBUF):
            pltpu.async_copy(hs_hbm.at[src_v.at[k]], bufs[k], gsem[k])

        @pl.loop(1, NGRP)
        def _(i):
            j0 = i * NBUF
            for k in range(NBUF):
                wait_gather(j0 - NBUF + k, k)
                pltpu.async_copy(bufs[k], acc_s.at[dst_v.at[j0 - NBUF + k]],
                                 ssem[k], add=True)
            for k in range(NBUF):
                wait_scatter(j0 - NBUF + k, k)
                pltpu.async_copy(hs_hbm.at[src_v.at[j0 + k]], bufs[k], gsem[k])

        for k in range(NBUF):
            wait_gather(CPT - NBUF + k, k)
            pltpu.async_copy(bufs[k], acc_s.at[dst_v.at[CPT - NBUF + k]],
                             ssem[k], add=True)
        for k in range(NBUF):
            wait_scatter(CPT - NBUF + k, k)

        plsc.subcore_barrier()
        pltpu.sync_copy(acc_s.at[pl.ds(base, RPT)],
                        out_hbm.at[cid, pl.ds(base, RPT)])

    return edge_kernel(hs, src_tiles, dst_tiles, zeros128)


def _dot(a, b):
    return jnp.dot(a, b, preferred_element_type=jnp.float32,
                   precision=lax.Precision.HIGHEST)


def _row_spec(cols):
    return pl.BlockSpec((MBLK, cols), lambda i: (i, 0))


def _full_spec(rows, cols):
    return pl.BlockSpec((rows, cols), lambda i: (0, 0))


def _tc_encode(x, enc_W, enc_b):
    def body(x_ref, w_ref, b_ref, o_ref):
        o_ref[...] = jnp.maximum(_dot(x_ref[...], w_ref[...]) + b_ref[...], 0.0)

    return pl.pallas_call(
        body,
        grid=(N // MBLK,),
        in_specs=[_row_spec(NFEAT), _full_spec(NFEAT, NHID), _full_spec(1, NHID)],
        out_specs=_row_spec(NHID),
        out_shape=jax.ShapeDtypeStruct((N, NHID), jnp.float32),
    )(x, enc_W, enc_b.reshape(1, NHID))


def _tc_scale_matmul(x0, conv_W, dega, degb):
    def body(x_ref, w_ref, da_ref, db_ref, o_ref):
        dinv = lax.rsqrt(da_ref[:, 0:1] + db_ref[:, 0:1] + 1.0)
        o_ref[...] = dinv * _dot(x_ref[...], w_ref[...])

    return pl.pallas_call(
        body,
        grid=(N // MBLK,),
        in_specs=[_row_spec(NHID), _full_spec(NHID, NHID),
                  _row_spec(16), _row_spec(16)],
        out_specs=_row_spec(NHID),
        out_shape=jax.ShapeDtypeStruct((N, NHID), jnp.float32),
    )(x0, conv_W, dega, degb)


def _tc_post_and_next(acc0, acc1, hs, x_prev, conv_b, conv_W, dega, degb):
    def body(a0_ref, a1_ref, hs_ref, xp_ref, b_ref, w_ref, da_ref, db_ref,
             x_ref, hsn_ref):
        dinv = lax.rsqrt(da_ref[:, 0:1] + db_ref[:, 0:1] + 1.0)
        agg = a0_ref[...] + a1_ref[...] + hs_ref[...]
        x_new = jnp.maximum(dinv * agg + b_ref[...] + xp_ref[...], 0.0)
        x_ref[...] = x_new
        hsn_ref[...] = dinv * _dot(x_new, w_ref[...])

    return pl.pallas_call(
        body,
        grid=(N // MBLK,),
        in_specs=[_row_spec(NHID), _row_spec(NHID), _row_spec(NHID),
                  _row_spec(NHID), _full_spec(1, NHID), _full_spec(NHID, NHID),
                  _row_spec(16), _row_spec(16)],
        out_specs=[_row_spec(NHID), _row_spec(NHID)],
        out_shape=[jax.ShapeDtypeStruct((N, NHID), jnp.float32),
                   jax.ShapeDtypeStruct((N, NHID), jnp.float32)],
    )(acc0, acc1, hs, x_prev, conv_b.reshape(1, NHID), conv_W, dega, degb)


def _tc_post_and_decode(acc0, acc1, hs, x_prev, conv_b, dec_W, dec_b, dega, degb):
    def body(a0_ref, a1_ref, hs_ref, xp_ref, b_ref, w_ref, db2_ref, da_ref,
             db_ref, o_ref):
        dinv = lax.rsqrt(da_ref[:, 0:1] + db_ref[:, 0:1] + 1.0)
        agg = a0_ref[...] + a1_ref[...] + hs_ref[...]
        x_new = jnp.maximum(dinv * agg + b_ref[...] + xp_ref[...], 0.0)
        o_ref[...] = _dot(x_new, w_ref[...]) + db2_ref[...]

    return pl.pallas_call(
        body,
        grid=(N // MBLK,),
        in_specs=[_row_spec(NHID), _row_spec(NHID), _row_spec(NHID),
                  _row_spec(NHID), _full_spec(1, NHID), _full_spec(NHID, NCLASS),
                  _full_spec(1, NCLASS), _row_spec(16), _row_spec(16)],
        out_specs=_row_spec(NCLASS),
        out_shape=jax.ShapeDtypeStruct((N, NCLASS), jnp.float32),
    )(acc0, acc1, hs, x_prev, conv_b.reshape(1, NHID), dec_W,
      dec_b.reshape(1, NCLASS), dega, degb)


def kernel(x, edge_index, enc_W, enc_b, conv_W, conv_b, dec_W, dec_b):
    src = edge_index[0]
    dst = edge_index[1]
    pad = EPAD - E
    srcp = jnp.concatenate([src, jnp.zeros((pad,), jnp.int32)])
    dstp = jnp.concatenate([dst, jnp.full((pad,), N, jnp.int32)])
    src_tiles = srcp.reshape(NW, CPT, CHUNK)
    dst_tiles = dstp.reshape(NW, CPT, CHUNK)

    zeros128 = jnp.zeros((NP, NHID), jnp.float32)
    ones_rows = jnp.ones((CHUNK, DEGW), jnp.float32)

    degs = _sc_degree(dst_tiles, ones_rows, zeros128)
    dega = degs[0, :N, :16]
    degb = degs[1, :N, :16]

    x0 = _tc_encode(x, enc_W, enc_b)
    hs1 = _tc_scale_matmul(x0, conv_W, dega, degb)

    acc = _sc_edge_pass(hs1, src_tiles, dst_tiles, zeros128)
    x1, hs2 = _tc_post_and_next(acc[0, :N], acc[1, :N], hs1, x0,
                                conv_b, conv_W, dega, degb)

    acc2 = _sc_edge_pass(hs2, src_tiles, dst_tiles, zeros128)
    out = _tc_post_and_decode(acc2[0, :N], acc2[1, :N], hs2, x1,
                              conv_b, dec_W, dec_b, dega, degb)
    return out


# speedup vs baseline: 1.3469x; 1.0013x over previous
import functools

import jax
import jax.numpy as jnp
from jax import lax
from jax.experimental import pallas as pl
from jax.experimental.pallas import tpu as pltpu
from jax.experimental.pallas import tpu_sc as plsc

N = 10000
E = 160000
NFEAT = 128
NHID = 128
NCLASS = 64

NC = 2
NS = 16
NW = NC * NS
CHUNK = 128
CPT = 40
C0 = 64
C1 = 16
TOTCH = NS * (C0 + C1)
EPAD = NW * CPT * CHUNK
NP = 10112
RPT = NP // NS

_MESH = plsc.VectorSubcoreMesh(core_axis_name="c", subcore_axis_name="s")

MBLK = 2000


DEGW = 128


def _sc_degree(dst_tiles, ones_rows, zeros_deg):
    @functools.partial(
        pl.kernel,
        out_type=jax.ShapeDtypeStruct((NC, NP, DEGW), jnp.float32),
        mesh=_MESH,
        scratch_types=[
            pltpu.VMEM((CPT, CHUNK), jnp.int32),
            pltpu.VMEM((CHUNK, DEGW), jnp.float32),
            pltpu.VMEM_SHARED((NP, DEGW), jnp.float32),
            pltpu.SemaphoreType.DMA,
            pltpu.SemaphoreType.DMA,
        ],
    )
    def deg_kernel(dst_hbm, ones_hbm, zeros_hbm, out_hbm, idx_v, ones_v, acc_s,
                   s0, s1):
        cid = lax.axis_index("c")
        sid = lax.axis_index("s")
        gtile = cid * NS + sid
        base = sid * RPT
        pltpu.sync_copy(zeros_hbm.at[pl.ds(base, RPT)], acc_s.at[pl.ds(base, RPT)])
        pltpu.sync_copy(ones_hbm, ones_v)
        pltpu.sync_copy(dst_hbm.at[pl.ds(gtile * CPT, CPT)], idx_v)
        plsc.subcore_barrier()

        @pl.loop(0, CPT)
        def _(j):
            pltpu.async_copy(ones_v, acc_s.at[idx_v.at[j]], s0, add=True)

        @pl.loop(0, CPT)
        def _(j):
            pltpu.make_async_copy(ones_v, acc_s.at[idx_v.at[j]], s0).wait()

        plsc.subcore_barrier()
        pltpu.sync_copy(acc_s.at[pl.ds(base, RPT)],
                        out_hbm.at[cid, pl.ds(base, RPT)])

    return deg_kernel(dst_tiles, ones_rows, zeros_deg)


def _sc_edge_pass(hs, src_flat, dst_flat, zeros128):
    NBUF = 2

    @functools.partial(
        pl.kernel,
        out_type=jax.ShapeDtypeStruct((NC, NP, NHID), jnp.float32),
        mesh=_MESH,
        scratch_types=(
            [pltpu.VMEM((C0, CHUNK), jnp.int32),
             pltpu.VMEM((C0, CHUNK), jnp.int32)]
            + [pltpu.VMEM((CHUNK, NHID), jnp.float32)] * NBUF
            + [pltpu.SemaphoreType.DMA] * (2 * NBUF)
            + [pltpu.VMEM_SHARED((NP, NHID), jnp.float32)]
        ),
    )
    def edge_kernel(hs_hbm, src_hbm, dst_hbm, zeros_hbm, out_hbm,
                    src_v, dst_v, *rest):
        bufs = rest[:NBUF]
        gsem = rest[NBUF:2 * NBUF]
        ssem = rest[2 * NBUF:3 * NBUF]
        acc_s = rest[3 * NBUF]
        cid = lax.axis_index("c")
        sid = lax.axis_index("s")
        base = sid * RPT
        pltpu.sync_copy(zeros_hbm.at[pl.ds(base, RPT)], acc_s.at[pl.ds(base, RPT)])

        def wait_gather(j, k):
            pltpu.make_async_copy(hs_hbm.at[src_v.at[j]], bufs[k], gsem[k]).wait()

        def wait_scatter(j, k):
            pltpu.make_async_copy(bufs[k], acc_s.at[dst_v.at[j]], ssem[k]).wait()

        def run(chunk0, cnt):
            assert cnt % NBUF == 0
            pltpu.sync_copy(src_hbm.at[pl.ds(chunk0, cnt)],
                            src_v.at[pl.ds(0, cnt)])
            pltpu.sync_copy(dst_hbm.at[pl.ds(chunk0, cnt)],
                            dst_v.at[pl.ds(0, cnt)])
            plsc.subcore_barrier()
            for k in range(NBUF):
                pltpu.async_copy(hs_hbm.at[src_v.at[k]], bufs[k], gsem[k])

            @pl.loop(1, cnt // NBUF)
            def _(i):
                j0 = i * NBUF
                for k in range(NBUF):
                    wait_gather(j0 - NBUF + k, k)
                    pltpu.async_copy(bufs[k], acc_s.at[dst_v.at[j0 - NBUF + k]],
                                     ssem[k], add=True)
                for k in range(NBUF):
                    wait_scatter(j0 - NBUF + k, k)
                    pltpu.async_copy(hs_hbm.at[src_v.at[j0 + k]], bufs[k], gsem[k])

            for k in range(NBUF):
                wait_gather(cnt - NBUF + k, k)
                pltpu.async_copy(bufs[k], acc_s.at[dst_v.at[cnt - NBUF + k]],
                                 ssem[k], add=True)
            for k in range(NBUF):
                wait_scatter(cnt - NBUF + k, k)

        @pl.when(cid == 0)
        def _():
            run(sid * C0, C0)

        @pl.when(cid == 1)
        def _():
            run(NS * C0 + sid * C1, C1)

        plsc.subcore_barrier()
        pltpu.sync_copy(acc_s.at[pl.ds(base, RPT)],
                        out_hbm.at[cid, pl.ds(base, RPT)])

    return edge_kernel(hs, src_flat, dst_flat, zeros128)


def _dot(a, b):
    return jnp.dot(a, b, preferred_element_type=jnp.float32,
                   precision=lax.Precision.HIGHEST)


def _row_spec(cols):
    return pl.BlockSpec((MBLK, cols), lambda i: (i, 0))


def _full_spec(rows, cols):
    return pl.BlockSpec((rows, cols), lambda i: (0, 0))


def _tc_encode(x, enc_W, enc_b):
    def body(x_ref, w_ref, b_ref, o_ref):
        o_ref[...] = jnp.maximum(_dot(x_ref[...], w_ref[...]) + b_ref[...], 0.0)

    return pl.pallas_call(
        body,
        grid=(N // MBLK,),
        in_specs=[_row_spec(NFEAT), _full_spec(NFEAT, NHID), _full_spec(1, NHID)],
        out_specs=_row_spec(NHID),
        out_shape=jax.ShapeDtypeStruct((N, NHID), jnp.float32),
    )(x, enc_W, enc_b.reshape(1, NHID))


def _tc_scale_matmul(x0, conv_W, dega, degb):
    def body(x_ref, w_ref, da_ref, db_ref, o_ref):
        dinv = lax.rsqrt(da_ref[:, 0:1] + db_ref[:, 0:1] + 1.0)
        o_ref[...] = dinv * _dot(x_ref[...], w_ref[...])

    return pl.pallas_call(
        body,
        grid=(N // MBLK,),
        in_specs=[_row_spec(NHID), _full_spec(NHID, NHID),
                  _row_spec(16), _row_spec(16)],
        out_specs=_row_spec(NHID),
        out_shape=jax.ShapeDtypeStruct((N, NHID), jnp.float32),
    )(x0, conv_W, dega, degb)


def _tc_post_and_next(acc0, acc1, hs, x_prev, conv_b, conv_W, dega, degb):
    def body(a0_ref, a1_ref, hs_ref, xp_ref, b_ref, w_ref, da_ref, db_ref,
             x_ref, hsn_ref):
        dinv = lax.rsqrt(da_ref[:, 0:1] + db_ref[:, 0:1] + 1.0)
        agg = a0_ref[...] + a1_ref[...] + hs_ref[...]
        x_new = jnp.maximum(dinv * agg + b_ref[...] + xp_ref[...], 0.0)
        x_ref[...] = x_new
        hsn_ref[...] = dinv * _dot(x_new, w_ref[...])

    return pl.pallas_call(
        body,
        grid=(N // MBLK,),
        in_specs=[_row_spec(NHID), _row_spec(NHID), _row_spec(NHID),
                  _row_spec(NHID), _full_spec(1, NHID), _full_spec(NHID, NHID),
                  _row_spec(16), _row_spec(16)],
        out_specs=[_row_spec(NHID), _row_spec(NHID)],
        out_shape=[jax.ShapeDtypeStruct((N, NHID), jnp.float32),
                   jax.ShapeDtypeStruct((N, NHID), jnp.float32)],
    )(acc0, acc1, hs, x_prev, conv_b.reshape(1, NHID), conv_W, dega, degb)


def _tc_post_and_decode(acc0, acc1, hs, x_prev, conv_b, dec_W, dec_b, dega, degb):
    def body(a0_ref, a1_ref, hs_ref, xp_ref, b_ref, w_ref, db2_ref, da_ref,
             db_ref, o_ref):
        dinv = lax.rsqrt(da_ref[:, 0:1] + db_ref[:, 0:1] + 1.0)
        agg = a0_ref[...] + a1_ref[...] + hs_ref[...]
        x_new = jnp.maximum(dinv * agg + b_ref[...] + xp_ref[...], 0.0)
        o_ref[...] = _dot(x_new, w_ref[...]) + db2_ref[...]

    return pl.pallas_call(
        body,
        grid=(N // MBLK,),
        in_specs=[_row_spec(NHID), _row_spec(NHID), _row_spec(NHID),
                  _row_spec(NHID), _full_spec(1, NHID), _full_spec(NHID, NCLASS),
                  _full_spec(1, NCLASS), _row_spec(16), _row_spec(16)],
        out_specs=_row_spec(NCLASS),
        out_shape=jax.ShapeDtypeStruct((N, NCLASS), jnp.float32),
    )(acc0, acc1, hs, x_prev, conv_b.reshape(1, NHID), dec_W,
      dec_b.reshape(1, NCLASS), dega, degb)


def kernel(x, edge_index, enc_W, enc_b, conv_W, conv_b, dec_W, dec_b):
    src = edge_index[0]
    dst = edge_index[1]
    pad = EPAD - E
    srcp = jnp.concatenate([src, jnp.zeros((pad,), jnp.int32)])
    dstp = jnp.concatenate([dst, jnp.full((pad,), N, jnp.int32)])
    src_flat = srcp.reshape(TOTCH, CHUNK)
    dst_flat = dstp.reshape(TOTCH, CHUNK)

    zeros128 = jnp.zeros((NP, NHID), jnp.float32)
    ones_rows = jnp.ones((CHUNK, DEGW), jnp.float32)

    degs = _sc_degree(dst_flat, ones_rows, zeros128)
    dega = degs[0, :N, :16]
    degb = degs[1, :N, :16]

    x0 = _tc_encode(x, enc_W, enc_b)
    hs1 = _tc_scale_matmul(x0, conv_W, dega, degb)

    acc = _sc_edge_pass(hs1, src_flat, dst_flat, zeros128)
    x1, hs2 = _tc_post_and_next(acc[0, :N], acc[1, :N], hs1, x0,
                                conv_b, conv_W, dega, degb)

    acc2 = _sc_edge_pass(hs2, src_flat, dst_flat, zeros128)
    out = _tc_post_and_decode(acc2[0, :N], acc2[1, :N], hs2, x1,
                              conv_b, dec_W, dec_b, dega, degb)
    return out
